# baseline (speedup 1.0000x reference)
import jax
import jax.numpy as jnp
from jax.experimental import pallas as pl
from jax.experimental.pallas import tpu as pltpu

BM = 400
H = BM // 2
CSTRIDE = 13


def _dot(a, b):
    return jax.lax.dot_general(
        a, b, (((1,), (0,)), ((), ())), preferred_element_type=jnp.float32,
        precision=jax.lax.Precision.DEFAULT)


def _body(ga_ref, gb_ref, x_ref, w1_ref, w2_ref, w02_ref, o_ref,
          t1_ref, cache_ref, acc_ref):
    p = pl.program_id(0)
    i = pl.program_id(1)
    cached = (i % CSTRIDE) == 0
    slot = i // CSTRIDE

    @pl.when(p == 0)
    def _phase0():
        t1a = _dot(ga_ref[...], x_ref[...])
        t1b = _dot(gb_ref[...], x_ref[...])
        t1_ref[pl.ds(i * BM, BM), :] = jnp.concatenate([t1a, t1b], axis=0)

        @pl.when(cached)
        def _fill_cache():
            cache_ref[pl.ds(slot * BM, H), :] = (
                ga_ref[...].astype(jnp.bfloat16))
            cache_ref[pl.ds(slot * BM + H, H), :] = (
                gb_ref[...].astype(jnp.bfloat16))

    @pl.when((p == 1) & cached)
    def _phase1_cached():
        acc_ref[0:H, :] = _dot(
            cache_ref[pl.ds(slot * BM, H), :].astype(jnp.float32),
            t1_ref[...])
        acc_ref[H:BM, :] = _dot(
            cache_ref[pl.ds(slot * BM + H, H), :].astype(jnp.float32),
            t1_ref[...])

    @pl.when((p == 1) & jnp.logical_not(cached))
    def _phase1_streamed():
        acc_ref[0:H, :] = _dot(ga_ref[...], t1_ref[...])
        acc_ref[H:BM, :] = _dot(gb_ref[...], t1_ref[...])

    @pl.when(p == 1)
    def _epilogue():
        t1i = t1_ref[pl.ds(i * BM, BM), :]
        xi = x_ref[pl.ds(i * BM, BM), :]
        o_ref[...] = (_dot(2.0 * acc_ref[...], w2_ref[...])
                      + _dot(t1i, w1_ref[...])
                      + _dot(xi, w02_ref[...]))


def kernel(x, gso, weight):
    n, in_size = x.shape
    out_size = weight.shape[2]
    nm = n // BM
    nc = (nm + CSTRIDE - 1) // CSTRIDE

    w0, w1, w2 = weight[0], weight[1], weight[2]
    w02 = w0 - w2

    def win_idx(off):
        def idx(pp, ii):
            c = (ii % CSTRIDE) == 0
            prev = jnp.where(ii == 0, nm - 1, ii - 1)
            blk = jnp.where((pp == 1) & c, prev, ii)
            return (2 * blk + off, 0)
        return idx

    full = pl.BlockSpec((n, in_size), lambda p, i: (0, 0))
    wspec = pl.BlockSpec((in_size, out_size), lambda p, i: (0, 0))
    fused = pl.pallas_call(
        _body,
        grid=(2, nm),
        in_specs=[
            pl.BlockSpec((H, n), win_idx(0)),
            pl.BlockSpec((H, n), win_idx(1)),
            full, wspec, wspec, wspec,
        ],
        out_specs=pl.BlockSpec((BM, out_size), lambda p, i: (i * p, 0)),
        out_shape=jax.ShapeDtypeStruct((n, out_size), jnp.float32),
        scratch_shapes=[
            pltpu.VMEM((n, in_size), jnp.float32),
            pltpu.VMEM((nc * BM, n), jnp.bfloat16),
            pltpu.VMEM((BM, out_size), jnp.float32),
        ],
        compiler_params=pltpu.CompilerParams(
            dimension_semantics=("arbitrary", "arbitrary"),
            vmem_limit_bytes=67108864,
        ),
        cost_estimate=pl.CostEstimate(
            flops=4 * n * n * in_size, bytes_accessed=2 * gso.size * 4,
            transcendentals=0),
    )
    return fused(gso, gso, x, w1, w2, w02)


# speedup vs baseline: 1.0006x; 1.0006x over previous
import jax
import jax.numpy as jnp
from jax.experimental import pallas as pl
from jax.experimental.pallas import tpu as pltpu

BM = 400
H = BM // 2
CSTRIDE = 13


def _dot(a, b):
    return jax.lax.dot_general(
        a, b, (((1,), (0,)), ((), ())), preferred_element_type=jnp.float32,
        precision=jax.lax.Precision.DEFAULT)


def _body(ga_ref, gb_ref, x_ref, w1_ref, w2_ref, w02_ref, o_ref,
          t1_ref, t1b16_ref, cache_ref, acc_ref):
    p = pl.program_id(0)
    i = pl.program_id(1)
    cached = (i % CSTRIDE) == 0
    slot = i // CSTRIDE

    @pl.when(p == 0)
    def _phase0():
        t1a = _dot(ga_ref[...], x_ref[...])
        t1b = _dot(gb_ref[...], x_ref[...])
        t1 = jnp.concatenate([t1a, t1b], axis=0)
        t1_ref[pl.ds(i * BM, BM), :] = t1
        t1b16_ref[pl.ds(i * BM, BM), :] = t1.astype(jnp.bfloat16)

        @pl.when(cached)
        def _fill_cache():
            cache_ref[pl.ds(slot * BM, H), :] = (
                ga_ref[...].astype(jnp.bfloat16))
            cache_ref[pl.ds(slot * BM + H, H), :] = (
                gb_ref[...].astype(jnp.bfloat16))

    @pl.when((p == 1) & cached)
    def _phase1_cached():
        acc_ref[0:H, :] = _dot(cache_ref[pl.ds(slot * BM, H), :],
                               t1b16_ref[...])
        acc_ref[H:BM, :] = _dot(cache_ref[pl.ds(slot * BM + H, H), :],
                                t1b16_ref[...])

    @pl.when((p == 1) & jnp.logical_not(cached))
    def _phase1_streamed():
        acc_ref[0:H, :] = _dot(ga_ref[...], t1_ref[...])
        acc_ref[H:BM, :] = _dot(gb_ref[...], t1_ref[...])

    @pl.when(p == 1)
    def _epilogue():
        t1i = t1_ref[pl.ds(i * BM, BM), :]
        xi = x_ref[pl.ds(i * BM, BM), :]
        o_ref[...] = (_dot(2.0 * acc_ref[...], w2_ref[...])
                      + _dot(t1i, w1_ref[...])
                      + _dot(xi, w02_ref[...]))


def kernel(x, gso, weight):
    n, in_size = x.shape
    out_size = weight.shape[2]
    nm = n // BM
    nc = (nm + CSTRIDE - 1) // CSTRIDE

    w0, w1, w2 = weight[0], weight[1], weight[2]
    w02 = w0 - w2

    def win_idx(off):
        def idx(pp, ii):
            c = (ii % CSTRIDE) == 0
            prev = jnp.where(ii == 0, nm - 1, ii - 1)
            blk = jnp.where((pp == 1) & c, prev, ii)
            return (2 * blk + off, 0)
        return idx

    full = pl.BlockSpec((n, in_size), lambda p, i: (0, 0))
    wspec = pl.BlockSpec((in_size, out_size), lambda p, i: (0, 0))
    fused = pl.pallas_call(
        _body,
        grid=(2, nm),
        in_specs=[
            pl.BlockSpec((H, n), win_idx(0)),
            pl.BlockSpec((H, n), win_idx(1)),
            full, wspec, wspec, wspec,
        ],
        out_specs=pl.BlockSpec((BM, out_size), lambda p, i: (i * p, 0)),
        out_shape=jax.ShapeDtypeStruct((n, out_size), jnp.float32),
        scratch_shapes=[
            pltpu.VMEM((n, in_size), jnp.float32),
            pltpu.VMEM((n, in_size), jnp.bfloat16),
            pltpu.VMEM((nc * BM, n), jnp.bfloat16),
            pltpu.VMEM((BM, out_size), jnp.float32),
        ],
        compiler_params=pltpu.CompilerParams(
            dimension_semantics=("arbitrary", "arbitrary"),
            vmem_limit_bytes=67108864,
        ),
        cost_estimate=pl.CostEstimate(
            flops=4 * n * n * in_size, bytes_accessed=2 * gso.size * 4,
            transcendentals=0),
    )
    return fused(gso, gso, x, w1, w2, w02)
